# baseline (speedup 1.0000x reference)
import jax
import jax.numpy as jnp
from jax.experimental import pallas as pl
from jax.experimental.pallas import tpu as pltpu

_NUM_CLASSES = 21
_THRESHOLD = 0.5
_NEG_POS = 3
_NEG_W = 1.5
_POS_W = 1.0
_VAR0 = 0.1
_VAR1 = 0.2
_P = 8732
_SPLIT = 2916
_J = 4
_R = _P // _J
_NR = 18
_LANES = 128
_RPAD = _NR * _LANES
_TR = _SPLIT // _J
_TNR = 6
_TRPAD = _TNR * _LANES
_EPS = 1e-10
_BIG = 1 << 30


def _phase1_kernel(conf_ref, confT_ref, loc_ref, pri_ref, tgt_ref,
                   bits_ref, ce_ref, s_ref, posf_ref, scal_ref):
  f32 = jnp.float32

  def unpack(ref2d, rows, groups, nr, rpad):
    t = jnp.transpose(ref2d, (1, 0))
    t = jnp.pad(t, ((0, 0), (0, rpad - rows)))
    sub = t.shape[0] // groups
    return t.reshape(groups * sub, nr, _LANES).reshape(groups, sub, nr,
                                                       _LANES)

  pri4 = unpack(pri_ref[...], _R, _J, _NR, _RPAD)
  loc4 = unpack(loc_ref[0], _R, _J, _NR, _RPAD)
  ct = unpack(conf_ref[0], _R, _J, _NR, _RPAD)
  ctT = unpack(confT_ref[0], _TR, _J, _TNR, _TRPAD)

  S = (_J, _NR, _LANES)
  ji = jax.lax.broadcasted_iota(jnp.int32, S, 0)
  ii = jax.lax.broadcasted_iota(jnp.int32, S, 1)
  li = jax.lax.broadcasted_iota(jnp.int32, S, 2)
  pact = ii * (_J * _LANES) + li * _J + ji
  valid = (ii * _LANES + li) < _R

  pcx = pri4[:, 0]
  pcy = pri4[:, 1]
  pw = pri4[:, 2]
  ph = pri4[:, 3]
  px0 = pcx - pw * 0.5
  py0 = pcy - ph * 0.5
  px1 = pcx + pw * 0.5
  py1 = pcy + ph * 0.5
  area_p = (px1 - px0) * (py1 - py0)

  t_x0 = [tgt_ref[0, t, 0] for t in range(8)]
  t_y0 = [tgt_ref[0, t, 1] for t in range(8)]
  t_x1 = [tgt_ref[0, t, 2] for t in range(8)]
  t_y1 = [tgt_ref[0, t, 3] for t in range(8)]
  t_lab = [tgt_ref[0, t, 4] for t in range(8)]

  bto = jnp.full(S, -1.0, f32)
  bti = jnp.zeros(S, jnp.int32)
  bp_idx = []
  for t in range(8):
    iw = jnp.maximum(jnp.minimum(t_x1[t], px1) - jnp.maximum(t_x0[t], px0), 0.0)
    ih = jnp.maximum(jnp.minimum(t_y1[t], py1) - jnp.maximum(t_y0[t], py0), 0.0)
    inter = iw * ih
    area_t = (t_x1[t] - t_x0[t]) * (t_y1[t] - t_y0[t])
    ov = inter / (area_t + area_p - inter)
    ov = jnp.where(valid, ov, -1.0)
    upd = ov > bto
    bti = jnp.where(upd, t, bti)
    bto = jnp.maximum(bto, ov)
    m = jnp.max(ov, keepdims=True).reshape(1, 1, 1)
    bp = jnp.min(jnp.where(ov == m, pact, _BIG), keepdims=True).reshape(1, 1, 1)
    bp_idx.append(bp)
  for t in range(8):
    sel = pact == bp_idx[t]
    bto = jnp.where(sel, 2.0, bto)
    bti = jnp.where(sel, t, bti)

  mx0 = jnp.full(S, t_x0[0], f32)
  my0 = jnp.full(S, t_y0[0], f32)
  mx1 = jnp.full(S, t_x1[0], f32)
  my1 = jnp.full(S, t_y1[0], f32)
  lab = jnp.full(S, t_lab[0], f32)
  for t in range(1, 8):
    sel = bti == t
    mx0 = jnp.where(sel, t_x0[t], mx0)
    my0 = jnp.where(sel, t_y0[t], my0)
    mx1 = jnp.where(sel, t_x1[t], mx1)
    my1 = jnp.where(sel, t_y1[t], my1)
    lab = jnp.where(sel, t_lab[t], lab)
  conf_t = jnp.where(bto < _THRESHOLD, 0, lab.astype(jnp.int32) + 1)
  pos = conf_t > 0
  posf = jnp.where(pos, 1.0, 0.0).astype(f32)

  g_cx = ((mx0 + mx1) * 0.5 - pcx) / (_VAR0 * pw)
  g_cy = ((my0 + my1) * 0.5 - pcy) / (_VAR0 * ph)
  g_w = jnp.log((mx1 - mx0) / pw) / _VAR1
  g_h = jnp.log((my1 - my0) / ph) / _VAR1
  loss_l = jnp.zeros((1, 1, 1), f32)
  for j, g in enumerate((g_cx, g_cy, g_w, g_h)):
    d = loc4[:, j] - g
    ad = jnp.abs(d)
    sl1 = jnp.where(ad < 1.0, 0.5 * d * d, ad - 0.5)
    loss_l = loss_l + jnp.sum(jnp.where(pos, sl1, 0.0),
                              keepdims=True).reshape(1, 1, 1)
  npos = jnp.sum(posf, keepdims=True).reshape(1, 1, 1)

  ci = jax.lax.broadcasted_iota(jnp.int32, (_J, _NUM_CLASSES, _NR, _LANES), 1)
  onehot = ci == conf_t[:, None]
  wv = jnp.where(ci == 0, _NEG_W, _POS_W).astype(f32)

  cmax = jnp.max(ct, axis=1)
  e1 = jnp.exp(ct - cmax[:, None])
  s1 = jnp.sum(e1, axis=1)
  lse1 = jnp.log(s1) + cmax
  conf_at = jnp.sum(jnp.where(onehot, ct, 0.0), axis=1)
  loss_c = lse1 - conf_at
  loss_c = jnp.where(pos, 0.0, loss_c)
  loss_c = jnp.maximum(loss_c, 0.0)
  bits = jnp.where(valid, jax.lax.bitcast_convert_type(loss_c, jnp.int32), -1)

  p3 = e1 / s1[:, None]
  pm = jnp.max(p3, axis=1)
  e2 = jnp.exp(p3 - pm[:, None])
  s2 = jnp.sum(e2, axis=1)
  lse2 = jnp.log(s2) + pm
  p_at = jnp.sum(jnp.where(onehot, p3, 0.0), axis=1)
  ce = lse2 - p_at

  logp = jnp.log(p3 + _EPS)
  s_full = jnp.sum(wv * (p3 + _EPS) * logp, axis=1)

  tmax = jnp.max(ctT, axis=1)
  et = jnp.exp(ctT - tmax[:, None])
  st = jnp.sum(et, axis=1)
  q3 = et / st[:, None]
  wvt = wv[:, :, :_TNR, :]
  s_a = jnp.sum(wvt * (q3 + _EPS) * logp[:, :, :_TNR, :], axis=1)
  head = (ii[:, :_TNR, :] * _LANES + li[:, :_TNR, :]) < _TR
  s_head = jnp.where(head, s_a, s_full[:, :_TNR, :])
  s_out = jnp.concatenate([s_head, s_full[:, _TNR:, :]], axis=1)

  bits_ref[0] = bits
  ce_ref[0] = ce
  s_ref[0] = s_out
  posf_ref[0] = posf
  lane = jax.lax.broadcasted_iota(jnp.int32, (1, _LANES), 1)
  scal_ref[0] = jnp.where(lane == 0, loss_l[0],
                          jnp.where(lane == 1, npos[0], 0.0))


def _phase2_kernel(bits_ref, ce_ref, s_ref, posf_ref, scal_ref, u_ref, out_ref):
  x = bits_ref[...]
  b = x.shape[0]
  npos = scal_ref[:, :, 1].reshape(b, 1, 1, 1)
  k = jnp.minimum(npos.astype(jnp.int32) * _NEG_POS, _P - 1)
  one = jnp.ones((), jnp.int32)
  zero = jnp.zeros((), jnp.int32)

  t_thr = jnp.zeros((b, 1, 1, 1), jnp.int32)
  for bit in range(30, -1, -1):
    cand = t_thr | (1 << bit)
    cnt = jnp.sum(jnp.where(x >= cand, one, zero), axis=(1, 2, 3),
                  keepdims=True)
    t_thr = jnp.where(cnt >= k, cand, t_thr)
  cgt = jnp.sum(jnp.where(x > t_thr, one, zero), axis=(1, 2, 3), keepdims=True)
  r = k - cgt
  tie = x == t_thr
  pact = (jax.lax.broadcasted_iota(jnp.int32, x.shape, 2) * (_J * _LANES)
          + jax.lax.broadcasted_iota(jnp.int32, x.shape, 3) * _J
          + jax.lax.broadcasted_iota(jnp.int32, x.shape, 1))
  idx_cut = jnp.zeros((b, 1, 1, 1), jnp.int32)
  for bit in range(13, -1, -1):
    cand = idx_cut + (1 << bit)
    f = jnp.sum(jnp.where(tie & (pact < cand), one, zero), axis=(1, 2, 3),
                keepdims=True)
    idx_cut = jnp.where(f <= r, cand, idx_cut)
  neg = (x > t_thr) | (tie & (pact < idx_cut))
  mask = neg | (posf_ref[...] > 0.0)

  hard = jnp.sum(jnp.where(mask, ce_ref[...], 0.0))
  soft = -jnp.sum(jnp.where(mask, s_ref[...], 0.0))
  loss_l = jnp.sum(scal_ref[:, :, 0])
  n = jnp.sum(scal_ref[:, :, 1])
  u = u_ref[0, 0]
  out_ref[0, 0] = (u * hard + (1.0 - u) * soft + loss_l) / n


def kernel(loc_data, conf_data, priors, locT, confT, targets, u):
  del locT
  b = loc_data.shape[0]
  f32 = jnp.float32

  conf_in = conf_data.reshape(b, _R, _J * _NUM_CLASSES)
  confT_in = confT.reshape(b, _TR, _J * _NUM_CLASSES)
  loc_in = loc_data.reshape(b, _R, _J * 4)
  pri_in = priors.reshape(_R, _J * 4)
  u2 = u.reshape(1, 1).astype(f32)

  bits, ce, s, posf, scal = pl.pallas_call(
      _phase1_kernel,
      grid=(b,),
      in_specs=[
          pl.BlockSpec((1, _R, _J * _NUM_CLASSES), lambda i: (i, 0, 0)),
          pl.BlockSpec((1, _TR, _J * _NUM_CLASSES), lambda i: (i, 0, 0)),
          pl.BlockSpec((1, _R, _J * 4), lambda i: (i, 0, 0)),
          pl.BlockSpec((_R, _J * 4), lambda i: (0, 0)),
          pl.BlockSpec((1, 8, 5), lambda i: (i, 0, 0),
                       memory_space=pltpu.MemorySpace.SMEM),
      ],
      out_specs=[
          pl.BlockSpec((1, _J, _NR, _LANES), lambda i: (i, 0, 0, 0)),
          pl.BlockSpec((1, _J, _NR, _LANES), lambda i: (i, 0, 0, 0)),
          pl.BlockSpec((1, _J, _NR, _LANES), lambda i: (i, 0, 0, 0)),
          pl.BlockSpec((1, _J, _NR, _LANES), lambda i: (i, 0, 0, 0)),
          pl.BlockSpec((1, 1, _LANES), lambda i: (i, 0, 0)),
      ],
      out_shape=[
          jax.ShapeDtypeStruct((b, _J, _NR, _LANES), jnp.int32),
          jax.ShapeDtypeStruct((b, _J, _NR, _LANES), f32),
          jax.ShapeDtypeStruct((b, _J, _NR, _LANES), f32),
          jax.ShapeDtypeStruct((b, _J, _NR, _LANES), f32),
          jax.ShapeDtypeStruct((b, 1, _LANES), f32),
      ],
  )(conf_in, confT_in, loc_in, pri_in, targets)

  out = pl.pallas_call(
      _phase2_kernel,
      in_specs=[
          pl.BlockSpec(memory_space=pltpu.MemorySpace.VMEM),
          pl.BlockSpec(memory_space=pltpu.MemorySpace.VMEM),
          pl.BlockSpec(memory_space=pltpu.MemorySpace.VMEM),
          pl.BlockSpec(memory_space=pltpu.MemorySpace.VMEM),
          pl.BlockSpec(memory_space=pltpu.MemorySpace.VMEM),
          pl.BlockSpec(memory_space=pltpu.MemorySpace.SMEM),
      ],
      out_specs=pl.BlockSpec(memory_space=pltpu.MemorySpace.SMEM),
      out_shape=jax.ShapeDtypeStruct((1, 1), f32),
  )(bits, ce, s, posf, scal, u2)
  return out[0, 0]


# speedup vs baseline: 1.9293x; 1.9293x over previous
import jax
import jax.numpy as jnp
from jax.experimental import pallas as pl
from jax.experimental.pallas import tpu as pltpu

_NUM_CLASSES = 21
_THRESHOLD = 0.5
_NEG_POS = 3
_NEG_W = 1.5
_POS_W = 1.0
_VAR0 = 0.1
_VAR1 = 0.2
_SPLIT = 2916
_P = 8732
_ROWS = 70
_LANES = 128
_PP = _ROWS * _LANES
_TROWS = 23
_CHUNK = 2184
_TCHUNK = 1464
_EPS = 1e-10
_BIG = 1 << 30


def _phase1_kernel(conf0, conf1, conf2, conf3, confT0, confT1,
                   loc_ref, pri_ref, tgt_ref,
                   bits_ref, ce_ref, s_ref, posf_ref, scal_ref):
  f32 = jnp.float32
  pcx = pri_ref[0]
  pcy = pri_ref[1]
  pw = pri_ref[2]
  ph = pri_ref[3]
  px0 = pcx - pw * 0.5
  py0 = pcy - ph * 0.5
  px1 = pcx + pw * 0.5
  py1 = pcy + ph * 0.5
  area_p = (px1 - px0) * (py1 - py0)
  flat = (jax.lax.broadcasted_iota(jnp.int32, (_ROWS, _LANES), 0) * _LANES
          + jax.lax.broadcasted_iota(jnp.int32, (_ROWS, _LANES), 1))
  valid = flat < _P

  t_x0 = [tgt_ref[0, t, 0] for t in range(8)]
  t_y0 = [tgt_ref[0, t, 1] for t in range(8)]
  t_x1 = [tgt_ref[0, t, 2] for t in range(8)]
  t_y1 = [tgt_ref[0, t, 3] for t in range(8)]
  t_lab = [tgt_ref[0, t, 4] for t in range(8)]

  bto = jnp.full((_ROWS, _LANES), -1.0, f32)
  bti = jnp.zeros((_ROWS, _LANES), jnp.int32)
  bp_idx = []
  for t in range(8):
    iw = jnp.maximum(jnp.minimum(t_x1[t], px1) - jnp.maximum(t_x0[t], px0), 0.0)
    ih = jnp.maximum(jnp.minimum(t_y1[t], py1) - jnp.maximum(t_y0[t], py0), 0.0)
    inter = iw * ih
    area_t = (t_x1[t] - t_x0[t]) * (t_y1[t] - t_y0[t])
    ov = inter / (area_t + area_p - inter)
    ov = jnp.where(valid, ov, -1.0)
    upd = ov > bto
    bti = jnp.where(upd, t, bti)
    bto = jnp.maximum(bto, ov)
    m = jnp.max(ov, keepdims=True).reshape(1, 1)
    bp = jnp.min(jnp.where(ov == m, flat, _BIG), keepdims=True).reshape(1, 1)
    bp_idx.append(bp)
  for t in range(8):
    sel = flat == bp_idx[t]
    bto = jnp.where(sel, 2.0, bto)
    bti = jnp.where(sel, t, bti)

  mx0 = jnp.full((_ROWS, _LANES), t_x0[0], f32)
  my0 = jnp.full((_ROWS, _LANES), t_y0[0], f32)
  mx1 = jnp.full((_ROWS, _LANES), t_x1[0], f32)
  my1 = jnp.full((_ROWS, _LANES), t_y1[0], f32)
  lab = jnp.full((_ROWS, _LANES), t_lab[0], f32)
  for t in range(1, 8):
    sel = bti == t
    mx0 = jnp.where(sel, t_x0[t], mx0)
    my0 = jnp.where(sel, t_y0[t], my0)
    mx1 = jnp.where(sel, t_x1[t], mx1)
    my1 = jnp.where(sel, t_y1[t], my1)
    lab = jnp.where(sel, t_lab[t], lab)
  conf_t = jnp.where(bto < _THRESHOLD, 0, lab.astype(jnp.int32) + 1)
  pos = conf_t > 0
  posf = jnp.where(pos, 1.0, 0.0).astype(f32)

  g_cx = ((mx0 + mx1) * 0.5 - pcx) / (_VAR0 * pw)
  g_cy = ((my0 + my1) * 0.5 - pcy) / (_VAR0 * ph)
  g_w = jnp.log((mx1 - mx0) / pw) / _VAR1
  g_h = jnp.log((my1 - my0) / ph) / _VAR1
  loss_l = jnp.zeros((1, 1), f32)
  for j, g in enumerate((g_cx, g_cy, g_w, g_h)):
    d = loc_ref[0, j] - g
    ad = jnp.abs(d)
    sl1 = jnp.where(ad < 1.0, 0.5 * d * d, ad - 0.5)
    loss_l = loss_l + jnp.sum(jnp.where(pos, sl1, 0.0), keepdims=True).reshape(1, 1)
  npos = jnp.sum(posf, keepdims=True).reshape(1, 1)

  c2 = jnp.concatenate(
      [jnp.transpose(r[0], (1, 0)) for r in (conf0, conf1, conf2, conf3)],
      axis=1)
  c3 = jnp.pad(c2, ((0, 0), (0, _PP - 4 * _CHUNK))).reshape(
      _NUM_CLASSES, _ROWS, _LANES)
  ci = jax.lax.broadcasted_iota(jnp.int32, (_NUM_CLASSES, _ROWS, _LANES), 0)
  onehot = ci == conf_t[None]
  wv = jnp.where(ci == 0, _NEG_W, _POS_W).astype(f32)

  cmax = jnp.max(c3, axis=0)
  e1 = jnp.exp(c3 - cmax[None])
  s1 = jnp.sum(e1, axis=0)
  lse1 = jnp.log(s1) + cmax
  conf_at = jnp.sum(jnp.where(onehot, c3, 0.0), axis=0)
  loss_c = lse1 - conf_at
  loss_c = jnp.where(pos, 0.0, loss_c)
  loss_c = jnp.maximum(loss_c, 0.0)
  bits = jnp.where(valid, jax.lax.bitcast_convert_type(loss_c, jnp.int32), -1)

  p3 = e1 / s1[None]
  pm = jnp.max(p3, axis=0)
  e2 = jnp.exp(p3 - pm[None])
  s2 = jnp.sum(e2, axis=0)
  lse2 = jnp.log(s2) + pm
  p_at = jnp.sum(jnp.where(onehot, p3, 0.0), axis=0)
  ce = lse2 - p_at

  logp = jnp.log(p3 + _EPS)
  s_full = jnp.sum(wv * (p3 + _EPS) * logp, axis=0)

  ct2 = jnp.concatenate(
      [jnp.transpose(r[0], (1, 0)) for r in (confT0, confT1)],
      axis=1)
  ct3 = jnp.pad(ct2, ((0, 0), (0, _TROWS * _LANES - 2 * _TCHUNK))).reshape(
      _NUM_CLASSES, _TROWS, _LANES)
  tmax = jnp.max(ct3, axis=0)
  et = jnp.exp(ct3 - tmax[None])
  st = jnp.sum(et, axis=0)
  q3 = et / st[None]
  wvt = wv[:, :_TROWS, :]
  s_a = jnp.sum(wvt * (q3 + _EPS) * logp[:, :_TROWS, :], axis=0)
  flat_a = flat[:_TROWS, :]
  s_head = jnp.where(flat_a < _SPLIT, s_a, s_full[:_TROWS, :])
  s_out = jnp.concatenate([s_head, s_full[_TROWS:, :]], axis=0)

  bits_ref[0] = bits
  ce_ref[0] = ce
  s_ref[0] = s_out
  posf_ref[0] = posf
  li = jax.lax.broadcasted_iota(jnp.int32, (1, _LANES), 1)
  scal_ref[0] = jnp.where(li == 0, loss_l, jnp.where(li == 1, npos, 0.0))


def _phase2_kernel(bits_ref, ce_ref, s_ref, posf_ref, scal_ref, u_ref, out_ref):
  x = bits_ref[...]
  b = x.shape[0]
  npos = scal_ref[:, :, 1].reshape(b, 1, 1)
  k = jnp.minimum(npos.astype(jnp.int32) * _NEG_POS, _P - 1)
  one = jnp.ones((), jnp.int32)
  zero = jnp.zeros((), jnp.int32)

  t_thr = jnp.zeros((b, 1, 1), jnp.int32)
  for bit in range(30, -1, -1):
    cand = t_thr | (1 << bit)
    cnt = jnp.sum(jnp.where(x >= cand, one, zero), axis=(1, 2), keepdims=True)
    t_thr = jnp.where(cnt >= k, cand, t_thr)
  cgt = jnp.sum(jnp.where(x > t_thr, one, zero), axis=(1, 2), keepdims=True)
  r = k - cgt
  tie = x == t_thr
  flat = (jax.lax.broadcasted_iota(jnp.int32, x.shape, 1) * _LANES
          + jax.lax.broadcasted_iota(jnp.int32, x.shape, 2))
  idx_cut = jnp.zeros((b, 1, 1), jnp.int32)
  for bit in range(13, -1, -1):
    cand = idx_cut + (1 << bit)
    f = jnp.sum(jnp.where(tie & (flat < cand), one, zero), axis=(1, 2),
                keepdims=True)
    idx_cut = jnp.where(f <= r, cand, idx_cut)
  neg = (x > t_thr) | (tie & (flat < idx_cut))
  mask = neg | (posf_ref[...] > 0.0)

  hard = jnp.sum(jnp.where(mask, ce_ref[...], 0.0))
  soft = -jnp.sum(jnp.where(mask, s_ref[...], 0.0))
  loss_l = jnp.sum(scal_ref[:, :, 0])
  n = jnp.sum(scal_ref[:, :, 1])
  u = u_ref[0, 0]
  out_ref[0, 0] = (u * hard + (1.0 - u) * soft + loss_l) / n


def kernel(loc_data, conf_data, priors, locT, confT, targets, u):
  del locT
  b = loc_data.shape[0]
  f32 = jnp.float32

  locR = jnp.pad(jnp.transpose(loc_data, (0, 2, 1)),
                 ((0, 0), (0, 0), (0, _PP - _P))).reshape(b, 4, _ROWS, _LANES)
  pri_pad = jnp.concatenate(
      [priors,
       jnp.tile(jnp.array([[0.5, 0.5, 1.0, 1.0]], f32), (_PP - _P, 1))],
      axis=0)
  priR = jnp.transpose(pri_pad, (1, 0)).reshape(4, _ROWS, _LANES)
  u2 = u.reshape(1, 1).astype(f32)

  def conf_spec(c):
    return pl.BlockSpec((1, _CHUNK, _NUM_CLASSES), lambda i, c=c: (i, c, 0))

  def confT_spec(c):
    return pl.BlockSpec((1, _TCHUNK, _NUM_CLASSES), lambda i, c=c: (i, c, 0))

  bits, ce, s, posf, scal = pl.pallas_call(
      _phase1_kernel,
      grid=(b,),
      in_specs=[
          conf_spec(0), conf_spec(1), conf_spec(2), conf_spec(3),
          confT_spec(0), confT_spec(1),
          pl.BlockSpec((1, 4, _ROWS, _LANES), lambda i: (i, 0, 0, 0)),
          pl.BlockSpec((4, _ROWS, _LANES), lambda i: (0, 0, 0)),
          pl.BlockSpec((1, 8, 5), lambda i: (i, 0, 0),
                       memory_space=pltpu.MemorySpace.SMEM),
      ],
      out_specs=[
          pl.BlockSpec((1, _ROWS, _LANES), lambda i: (i, 0, 0)),
          pl.BlockSpec((1, _ROWS, _LANES), lambda i: (i, 0, 0)),
          pl.BlockSpec((1, _ROWS, _LANES), lambda i: (i, 0, 0)),
          pl.BlockSpec((1, _ROWS, _LANES), lambda i: (i, 0, 0)),
          pl.BlockSpec((1, 1, _LANES), lambda i: (i, 0, 0)),
      ],
      out_shape=[
          jax.ShapeDtypeStruct((b, _ROWS, _LANES), jnp.int32),
          jax.ShapeDtypeStruct((b, _ROWS, _LANES), f32),
          jax.ShapeDtypeStruct((b, _ROWS, _LANES), f32),
          jax.ShapeDtypeStruct((b, _ROWS, _LANES), f32),
          jax.ShapeDtypeStruct((b, 1, _LANES), f32),
      ],
  )(conf_data, conf_data, conf_data, conf_data, confT, confT,
    locR, priR, targets)

  out = pl.pallas_call(
      _phase2_kernel,
      in_specs=[
          pl.BlockSpec(memory_space=pltpu.MemorySpace.VMEM),
          pl.BlockSpec(memory_space=pltpu.MemorySpace.VMEM),
          pl.BlockSpec(memory_space=pltpu.MemorySpace.VMEM),
          pl.BlockSpec(memory_space=pltpu.MemorySpace.VMEM),
          pl.BlockSpec(memory_space=pltpu.MemorySpace.VMEM),
          pl.BlockSpec(memory_space=pltpu.MemorySpace.SMEM),
      ],
      out_specs=pl.BlockSpec(memory_space=pltpu.MemorySpace.SMEM),
      out_shape=jax.ShapeDtypeStruct((1, 1), f32),
  )(bits, ce, s, posf, scal, u2)
  return out[0, 0]


# speedup vs baseline: 2.4327x; 1.2609x over previous
import jax
import jax.numpy as jnp
from jax.experimental import pallas as pl
from jax.experimental.pallas import tpu as pltpu

_NUM_CLASSES = 21
_THRESHOLD = 0.5
_NEG_POS = 3
_NEG_W = 1.5
_POS_W = 1.0
_VAR0 = 0.1
_VAR1 = 0.2
_SPLIT = 2916
_P = 8732
_ROWS = 70
_LANES = 128
_PP = _ROWS * _LANES
_TROWS = 23
_EPS = 1e-10
_BIG = 1 << 30


def _phase1_kernel(conf_ref, confT_ref, loc_ref, pri_ref, tgt_ref,
                   bits_ref, ce_ref, s_ref, posf_ref, scal_ref):
  f32 = jnp.float32
  pcx = pri_ref[0]
  pcy = pri_ref[1]
  pw = pri_ref[2]
  ph = pri_ref[3]
  px0 = pcx - pw * 0.5
  py0 = pcy - ph * 0.5
  px1 = pcx + pw * 0.5
  py1 = pcy + ph * 0.5
  area_p = (px1 - px0) * (py1 - py0)
  flat = (jax.lax.broadcasted_iota(jnp.int32, (_ROWS, _LANES), 0) * _LANES
          + jax.lax.broadcasted_iota(jnp.int32, (_ROWS, _LANES), 1))
  valid = flat < _P

  t_x0 = [tgt_ref[0, t, 0] for t in range(8)]
  t_y0 = [tgt_ref[0, t, 1] for t in range(8)]
  t_x1 = [tgt_ref[0, t, 2] for t in range(8)]
  t_y1 = [tgt_ref[0, t, 3] for t in range(8)]
  t_lab = [tgt_ref[0, t, 4] for t in range(8)]

  bto = jnp.full((_ROWS, _LANES), -1.0, f32)
  bti = jnp.zeros((_ROWS, _LANES), jnp.int32)
  bp_idx = []
  for t in range(8):
    iw = jnp.maximum(jnp.minimum(t_x1[t], px1) - jnp.maximum(t_x0[t], px0), 0.0)
    ih = jnp.maximum(jnp.minimum(t_y1[t], py1) - jnp.maximum(t_y0[t], py0), 0.0)
    inter = iw * ih
    area_t = (t_x1[t] - t_x0[t]) * (t_y1[t] - t_y0[t])
    ov = inter / (area_t + area_p - inter)
    ov = jnp.where(valid, ov, -1.0)
    upd = ov > bto
    bti = jnp.where(upd, t, bti)
    bto = jnp.maximum(bto, ov)
    m = jnp.max(ov, keepdims=True).reshape(1, 1)
    bp = jnp.min(jnp.where(ov == m, flat, _BIG), keepdims=True).reshape(1, 1)
    bp_idx.append(bp)
  for t in range(8):
    sel = flat == bp_idx[t]
    bto = jnp.where(sel, 2.0, bto)
    bti = jnp.where(sel, t, bti)

  mx0 = jnp.full((_ROWS, _LANES), t_x0[0], f32)
  my0 = jnp.full((_ROWS, _LANES), t_y0[0], f32)
  mx1 = jnp.full((_ROWS, _LANES), t_x1[0], f32)
  my1 = jnp.full((_ROWS, _LANES), t_y1[0], f32)
  lab = jnp.full((_ROWS, _LANES), t_lab[0], f32)
  for t in range(1, 8):
    sel = bti == t
    mx0 = jnp.where(sel, t_x0[t], mx0)
    my0 = jnp.where(sel, t_y0[t], my0)
    mx1 = jnp.where(sel, t_x1[t], mx1)
    my1 = jnp.where(sel, t_y1[t], my1)
    lab = jnp.where(sel, t_lab[t], lab)
  conf_t = jnp.where(bto < _THRESHOLD, 0, lab.astype(jnp.int32) + 1)
  pos = conf_t > 0
  posf = jnp.where(pos, 1.0, 0.0).astype(f32)

  g_cx = ((mx0 + mx1) * 0.5 - pcx) / (_VAR0 * pw)
  g_cy = ((my0 + my1) * 0.5 - pcy) / (_VAR0 * ph)
  g_w = jnp.log((mx1 - mx0) / pw) / _VAR1
  g_h = jnp.log((my1 - my0) / ph) / _VAR1
  loss_l = jnp.zeros((1, 1), f32)
  for j, g in enumerate((g_cx, g_cy, g_w, g_h)):
    d = loc_ref[0, j] - g
    ad = jnp.abs(d)
    sl1 = jnp.where(ad < 1.0, 0.5 * d * d, ad - 0.5)
    loss_l = loss_l + jnp.sum(jnp.where(pos, sl1, 0.0), keepdims=True).reshape(1, 1)
  npos = jnp.sum(posf, keepdims=True).reshape(1, 1)

  c3 = conf_ref[0]
  ci = jax.lax.broadcasted_iota(jnp.int32, (_NUM_CLASSES, _ROWS, _LANES), 0)
  onehot = ci == conf_t[None]
  wv = jnp.where(ci == 0, _NEG_W, _POS_W).astype(f32)

  cmax = jnp.max(c3, axis=0)
  e1 = jnp.exp(c3 - cmax[None])
  s1 = jnp.sum(e1, axis=0)
  lse1 = jnp.log(s1) + cmax
  conf_at = jnp.sum(jnp.where(onehot, c3, 0.0), axis=0)
  loss_c = lse1 - conf_at
  loss_c = jnp.where(pos, 0.0, loss_c)
  loss_c = jnp.maximum(loss_c, 0.0)
  bits = jnp.where(valid, jax.lax.bitcast_convert_type(loss_c, jnp.int32), -1)

  p3 = e1 / s1[None]
  pm = jnp.max(p3, axis=0)
  e2 = jnp.exp(p3 - pm[None])
  s2 = jnp.sum(e2, axis=0)
  lse2 = jnp.log(s2) + pm
  p_at = jnp.sum(jnp.where(onehot, p3, 0.0), axis=0)
  ce = lse2 - p_at

  logp = jnp.log(p3 + _EPS)
  s_full = jnp.sum(wv * (p3 + _EPS) * logp, axis=0)

  ct3 = confT_ref[0]
  tmax = jnp.max(ct3, axis=0)
  et = jnp.exp(ct3 - tmax[None])
  st = jnp.sum(et, axis=0)
  q3 = et / st[None]
  wvt = wv[:, :_TROWS, :]
  s_a = jnp.sum(wvt * (q3 + _EPS) * logp[:, :_TROWS, :], axis=0)
  flat_a = flat[:_TROWS, :]
  s_head = jnp.where(flat_a < _SPLIT, s_a, s_full[:_TROWS, :])
  s_out = jnp.concatenate([s_head, s_full[_TROWS:, :]], axis=0)

  bits_ref[0] = bits
  ce_ref[0] = ce
  s_ref[0] = s_out
  posf_ref[0] = posf
  li = jax.lax.broadcasted_iota(jnp.int32, (1, _LANES), 1)
  scal_ref[0] = jnp.where(li == 0, loss_l, jnp.where(li == 1, npos, 0.0))


def _phase2_kernel(bits_ref, ce_ref, s_ref, posf_ref, scal_ref, u_ref, out_ref):
  x = bits_ref[...]
  b = x.shape[0]
  npos = scal_ref[:, :, 1].reshape(b, 1, 1)
  k = jnp.minimum(npos.astype(jnp.int32) * _NEG_POS, _P - 1)
  one = jnp.ones((), jnp.int32)
  zero = jnp.zeros((), jnp.int32)

  t_thr = jnp.zeros((b, 1, 1), jnp.int32)
  for bit in range(30, -1, -1):
    cand = t_thr | (1 << bit)
    cnt = jnp.sum(jnp.where(x >= cand, one, zero), axis=(1, 2), keepdims=True)
    t_thr = jnp.where(cnt >= k, cand, t_thr)
  cgt = jnp.sum(jnp.where(x > t_thr, one, zero), axis=(1, 2), keepdims=True)
  r = k - cgt
  tie = x == t_thr
  flat = (jax.lax.broadcasted_iota(jnp.int32, x.shape, 1) * _LANES
          + jax.lax.broadcasted_iota(jnp.int32, x.shape, 2))
  idx_cut = jnp.zeros((b, 1, 1), jnp.int32)
  for bit in range(13, -1, -1):
    cand = idx_cut + (1 << bit)
    f = jnp.sum(jnp.where(tie & (flat < cand), one, zero), axis=(1, 2),
                keepdims=True)
    idx_cut = jnp.where(f <= r, cand, idx_cut)
  neg = (x > t_thr) | (tie & (flat < idx_cut))
  mask = neg | (posf_ref[...] > 0.0)

  hard = jnp.sum(jnp.where(mask, ce_ref[...], 0.0))
  soft = -jnp.sum(jnp.where(mask, s_ref[...], 0.0))
  loss_l = jnp.sum(scal_ref[:, :, 0])
  n = jnp.sum(scal_ref[:, :, 1])
  u = u_ref[0, 0]
  out_ref[0, 0] = (u * hard + (1.0 - u) * soft + loss_l) / n


def kernel(loc_data, conf_data, priors, locT, confT, targets, u):
  del locT
  b = loc_data.shape[0]
  f32 = jnp.float32

  locR = jnp.pad(jnp.transpose(loc_data, (0, 2, 1)),
                 ((0, 0), (0, 0), (0, _PP - _P))).reshape(b, 4, _ROWS, _LANES)
  pri_pad = jnp.concatenate(
      [priors,
       jnp.tile(jnp.array([[0.5, 0.5, 1.0, 1.0]], f32), (_PP - _P, 1))],
      axis=0)
  priR = jnp.transpose(pri_pad, (1, 0)).reshape(4, _ROWS, _LANES)
  u2 = u.reshape(1, 1).astype(f32)

  confR = jnp.pad(jnp.transpose(conf_data, (0, 2, 1)),
                  ((0, 0), (0, 0), (0, _PP - _P))).reshape(
                      b, _NUM_CLASSES, _ROWS, _LANES)
  confTR = jnp.pad(jnp.transpose(confT, (0, 2, 1)),
                   ((0, 0), (0, 0), (0, _TROWS * _LANES - _SPLIT))).reshape(
                       b, _NUM_CLASSES, _TROWS, _LANES)

  bits, ce, s, posf, scal = pl.pallas_call(
      _phase1_kernel,
      grid=(b,),
      in_specs=[
          pl.BlockSpec((1, _NUM_CLASSES, _ROWS, _LANES),
                       lambda i: (i, 0, 0, 0)),
          pl.BlockSpec((1, _NUM_CLASSES, _TROWS, _LANES),
                       lambda i: (i, 0, 0, 0)),
          pl.BlockSpec((1, 4, _ROWS, _LANES), lambda i: (i, 0, 0, 0)),
          pl.BlockSpec((4, _ROWS, _LANES), lambda i: (0, 0, 0)),
          pl.BlockSpec((1, 8, 5), lambda i: (i, 0, 0),
                       memory_space=pltpu.MemorySpace.SMEM),
      ],
      out_specs=[
          pl.BlockSpec((1, _ROWS, _LANES), lambda i: (i, 0, 0)),
          pl.BlockSpec((1, _ROWS, _LANES), lambda i: (i, 0, 0)),
          pl.BlockSpec((1, _ROWS, _LANES), lambda i: (i, 0, 0)),
          pl.BlockSpec((1, _ROWS, _LANES), lambda i: (i, 0, 0)),
          pl.BlockSpec((1, 1, _LANES), lambda i: (i, 0, 0)),
      ],
      out_shape=[
          jax.ShapeDtypeStruct((b, _ROWS, _LANES), jnp.int32),
          jax.ShapeDtypeStruct((b, _ROWS, _LANES), f32),
          jax.ShapeDtypeStruct((b, _ROWS, _LANES), f32),
          jax.ShapeDtypeStruct((b, _ROWS, _LANES), f32),
          jax.ShapeDtypeStruct((b, 1, _LANES), f32),
      ],
  )(confR, confTR, locR, priR, targets)

  out = pl.pallas_call(
      _phase2_kernel,
      in_specs=[
          pl.BlockSpec(memory_space=pltpu.MemorySpace.VMEM),
          pl.BlockSpec(memory_space=pltpu.MemorySpace.VMEM),
          pl.BlockSpec(memory_space=pltpu.MemorySpace.VMEM),
          pl.BlockSpec(memory_space=pltpu.MemorySpace.VMEM),
          pl.BlockSpec(memory_space=pltpu.MemorySpace.VMEM),
          pl.BlockSpec(memory_space=pltpu.MemorySpace.SMEM),
      ],
      out_specs=pl.BlockSpec(memory_space=pltpu.MemorySpace.SMEM),
      out_shape=jax.ShapeDtypeStruct((1, 1), f32),
  )(bits, ce, s, posf, scal, u2)
  return out[0, 0]
